# baseline (speedup 1.0000x reference)
import functools

import jax
import jax.numpy as jnp
from jax import lax
from jax.experimental import pallas as pl
from jax.experimental.pallas import tpu as pltpu
from jax.experimental.pallas import tpu_sc as plsc

B, N, M, D, H, K = 2, 2048, 2048, 384, 6, 8
DK = D // H
BLK = 256
QBLK = 512
SC_CORES, SC_SUBCORES = 2, 16
NW = SC_CORES * SC_SUBCORES
GCHUNK = 128

_f32 = jnp.float32


def _ln(x, g, b, eps=1e-5):
    mu = jnp.mean(x, axis=-1, keepdims=True)
    xc = x - mu
    var = jnp.mean(xc * xc, axis=-1, keepdims=True)
    return xc * lax.rsqrt(var + eps) * g + b


def _dot(a, b):
    return jnp.dot(a.astype(jnp.bfloat16), b.astype(jnp.bfloat16),
                   preferred_element_type=_f32)


def _pre_body(qf_ref, g_ref, b_ref, wqkv_ref, wp1_ref, bp1_ref,
              qkv_ref, p1_ref, t1_ref):
    nf = _ln(qf_ref[0], g_ref[0], b_ref[0])
    qkv = _dot(nf, wqkv_ref[...])
    for j in range(3 * H):
        qkv_ref[0, j] = qkv[:, j * DK:(j + 1) * DK]
    wa = wp1_ref[:D]
    wb = wp1_ref[D:]
    p1_ref[0] = _dot(nf, wa)
    t1_ref[0] = _dot(nf, wb - wa) + bp1_ref[0]


def _pre_call(qf, g, b, wqkv, wp1, bp1):
    return pl.pallas_call(
        _pre_body,
        grid=(B, N // BLK),
        in_specs=[
            pl.BlockSpec((1, BLK, D), lambda b_, i: (b_, i, 0)),
            pl.BlockSpec((1, D), lambda b_, i: (0, 0)),
            pl.BlockSpec((1, D), lambda b_, i: (0, 0)),
            pl.BlockSpec((D, 3 * D), lambda b_, i: (0, 0)),
            pl.BlockSpec((2 * D, D), lambda b_, i: (0, 0)),
            pl.BlockSpec((1, D), lambda b_, i: (0, 0)),
        ],
        out_specs=[
            pl.BlockSpec((1, 3 * H, BLK, DK), lambda b_, i: (b_, 0, i, 0)),
            pl.BlockSpec((1, BLK, D), lambda b_, i: (b_, i, 0)),
            pl.BlockSpec((1, BLK, D), lambda b_, i: (b_, i, 0)),
        ],
        out_shape=[
            jax.ShapeDtypeStruct((B, 3 * H, N, DK), _f32),
            jax.ShapeDtypeStruct((B, N, D), _f32),
            jax.ShapeDtypeStruct((B, N, D), _f32),
        ],
        compiler_params=pltpu.CompilerParams(
            dimension_semantics=("parallel", "parallel")),
    )(qf, g, b, wqkv, wp1, bp1)


def _keys_body(kf_ref, g_ref, b_ref, wk_ref, wv_ref, wp2_ref, kv_ref, p2_ref):
    nk = _ln(kf_ref[0], g_ref[0], b_ref[0])
    k2 = _dot(nk, wk_ref[...])
    v2 = _dot(nk, wv_ref[...])
    for h in range(H):
        kv_ref[0, h] = k2[:, h * DK:(h + 1) * DK]
        kv_ref[0, H + h] = v2[:, h * DK:(h + 1) * DK]
    p2_ref[0] = _dot(nk, wp2_ref[:D])


def _keys_call(kf, g, b, wk, wv, wp2):
    return pl.pallas_call(
        _keys_body,
        grid=(B, M // BLK),
        in_specs=[
            pl.BlockSpec((1, BLK, D), lambda b_, i: (b_, i, 0)),
            pl.BlockSpec((1, D), lambda b_, i: (0, 0)),
            pl.BlockSpec((1, D), lambda b_, i: (0, 0)),
            pl.BlockSpec((D, D), lambda b_, i: (0, 0)),
            pl.BlockSpec((D, D), lambda b_, i: (0, 0)),
            pl.BlockSpec((2 * D, D), lambda b_, i: (0, 0)),
        ],
        out_specs=[
            pl.BlockSpec((1, 2 * H, BLK, DK), lambda b_, i: (b_, 0, i, 0)),
            pl.BlockSpec((1, BLK, D), lambda b_, i: (b_, i, 0)),
        ],
        out_shape=[
            jax.ShapeDtypeStruct((B, 2 * H, M, DK), _f32),
            jax.ShapeDtypeStruct((B, M, D), _f32),
        ],
        compiler_params=pltpu.CompilerParams(
            dimension_semantics=("parallel", "parallel")),
    )(kf, g, b, wk, wv, wp2)


def _attn_body(q_ref, k_ref, v_ref, o_ref):
    q = q_ref[0, 0].astype(jnp.bfloat16)
    k = k_ref[0, 0].astype(jnp.bfloat16)
    s = lax.dot_general(q, k, (((1,), (1,)), ((), ())),
                        preferred_element_type=_f32) * (DK ** -0.5)
    p = jax.nn.softmax(s, axis=-1)
    o_ref[0, 0] = _dot(p, v_ref[0, 0])


def _attn_call(q_arr, kv_arr, k_off, v_off, nkv):
    return pl.pallas_call(
        _attn_body,
        grid=(B, N // QBLK, H),
        in_specs=[
            pl.BlockSpec((1, 1, QBLK, DK), lambda b_, i, h: (b_, h, i, 0)),
            pl.BlockSpec((1, 1, nkv, DK),
                         lambda b_, i, h: (b_, h + k_off, 0, 0)),
            pl.BlockSpec((1, 1, nkv, DK),
                         lambda b_, i, h: (b_, h + v_off, 0, 0)),
        ],
        out_specs=pl.BlockSpec((1, 1, QBLK, DK),
                               lambda b_, i, h: (b_, h, i, 0)),
        out_shape=jax.ShapeDtypeStruct((B, H, N, DK), _f32),
        compiler_params=pltpu.CompilerParams(
            dimension_semantics=("parallel", "parallel", "arbitrary")),
    )(q_arr, kv_arr, kv_arr)


def _topk_body(qc_ref, kc_ref, idx_ref, *, nkeys):
    b_ = pl.program_id(0)
    q = qc_ref[0]
    kx = kc_ref[0]
    qq = jnp.zeros((BLK, 1), _f32)
    kk = jnp.zeros((1, nkeys), _f32)
    for c in range(3):
        qc_col = q[:, c:c + 1]
        kc_row = kx[c:c + 1, :]
        qq = qq + qc_col * qc_col
        kk = kk + kc_row * kc_row
    qk = jnp.dot(q.astype(jnp.bfloat16), kx.astype(jnp.bfloat16),
                 preferred_element_type=_f32)
    d2 = qq + kk - 2.0 * qk
    iota = lax.broadcasted_iota(jnp.int32, (BLK, nkeys), 1)
    cols = []
    for _ in range(K):
        m = jnp.min(d2, axis=1, keepdims=True)
        sel = jnp.min(jnp.where(d2 <= m, iota, nkeys), axis=1, keepdims=True)
        cols.append(sel)
        d2 = jnp.where(iota == sel, jnp.float32(jnp.inf), d2)
    idx_ref[0] = jnp.concatenate(cols, axis=1) + b_ * nkeys


def _topk_call(qct, kcoord, nkeys):
    body = functools.partial(_topk_body, nkeys=nkeys)
    return pl.pallas_call(
        body,
        grid=(B, N // BLK),
        in_specs=[
            pl.BlockSpec((1, BLK, 3), lambda b_, i: (b_, i, 0)),
            pl.BlockSpec((1, 3, nkeys), lambda b_, i: (b_, 0, 0)),
        ],
        out_specs=pl.BlockSpec((1, BLK, K), lambda b_, i: (b_, i, 0)),
        out_shape=jax.ShapeDtypeStruct((B, N, K), jnp.int32),
        compiler_params=pltpu.CompilerParams(
            dimension_semantics=("parallel", "parallel")),
    )(qct, kcoord)


@functools.lru_cache(maxsize=None)
def _make_sc_gather():
    mesh = plsc.VectorSubcoreMesh(core_axis_name="c", subcore_axis_name="s")

    @functools.partial(
        pl.kernel,
        out_type=jax.ShapeDtypeStruct((B * N * K, D), _f32),
        mesh=mesh,
        scratch_types=[
            pltpu.VMEM((GCHUNK,), jnp.int32),
            pltpu.VMEM((GCHUNK, D), _f32),
            pltpu.SemaphoreType.DMA,
        ],
    )
    def sc_gather(table_hbm, idx_hbm, out_hbm, idx_v, rows_v, sem):
        wid = lax.axis_index("s") * SC_CORES + lax.axis_index("c")
        per_w = (B * N * K) // NW
        base = wid * per_w
        for c in range(per_w // GCHUNK):
            off = base + c * GCHUNK
            pltpu.sync_copy(idx_hbm.at[pl.ds(off, GCHUNK)], idx_v)
            pltpu.async_copy(table_hbm.at[idx_v], rows_v, sem).wait()
            pltpu.sync_copy(rows_v, out_hbm.at[pl.ds(off, GCHUNK)])

    return sc_gather


def _gather_rows(table, idx):
    return _make_sc_gather()(table, idx)


def _combine_body(g_ref, t_ref, out_ref):
    t = t_ref[0]
    acc = None
    for k in range(K):
        x = g_ref[0, :, k, :] + t
        y = jnp.maximum(x, 0.2 * x)
        acc = y if acc is None else jnp.maximum(acc, y)
    out_ref[0] = acc


def _combine_call(gathered, t):
    return pl.pallas_call(
        _combine_body,
        grid=(B, N // BLK),
        in_specs=[
            pl.BlockSpec((1, BLK, K, D), lambda b_, i: (b_, i, 0, 0)),
            pl.BlockSpec((1, BLK, D), lambda b_, i: (b_, i, 0)),
        ],
        out_specs=pl.BlockSpec((1, BLK, D), lambda b_, i: (b_, i, 0)),
        out_shape=jax.ShapeDtypeStruct((B, N, D), _f32),
        compiler_params=pltpu.CompilerParams(
            dimension_semantics=("parallel", "parallel")),
    )(gathered, t)


def _mix1_body(attn_ref, geom_ref, qf_ref, wattn_ref, battn_ref, wsm_ref,
               bsm_ref, gcq_ref, bcq_ref, wq_ref, wp2_ref, bp2_ref,
               qfeat_ref, q2_ref, t2_ref):
    am = jnp.concatenate([attn_ref[0, h] for h in range(H)], axis=-1)
    a = _dot(am, wattn_ref[...]) + battn_ref[0]
    a2 = _dot(a, wsm_ref[:D]) + _dot(geom_ref[0], wsm_ref[D:]) + bsm_ref[0]
    qfeat = a2 + qf_ref[0]
    qfeat_ref[0] = qfeat
    nq = _ln(qfeat, gcq_ref[0], bcq_ref[0])
    q2 = _dot(nq, wq_ref[...])
    for h in range(H):
        q2_ref[0, h] = q2[:, h * DK:(h + 1) * DK]
    wa = wp2_ref[:D]
    wb = wp2_ref[D:]
    t2_ref[0] = _dot(nq, wb - wa) + bp2_ref[0]


def _mix1_call(attn_raw, geom, qf, wattn, battn, wsm, bsm, gcq, bcq, wq,
               wp2, bp2):
    tok = pl.BlockSpec((1, BLK, D), lambda b_, i: (b_, i, 0))
    hd = pl.BlockSpec((1, H, BLK, DK), lambda b_, i: (b_, 0, i, 0))
    vec = pl.BlockSpec((1, D), lambda b_, i: (0, 0))
    sq = pl.BlockSpec((D, D), lambda b_, i: (0, 0))
    dbl = pl.BlockSpec((2 * D, D), lambda b_, i: (0, 0))
    return pl.pallas_call(
        _mix1_body,
        grid=(B, N // BLK),
        in_specs=[hd, tok, tok, sq, vec, dbl, vec, vec, vec, sq, dbl, vec],
        out_specs=[tok, hd, tok],
        out_shape=[jax.ShapeDtypeStruct((B, N, D), _f32),
                   jax.ShapeDtypeStruct((B, H, N, DK), _f32),
                   jax.ShapeDtypeStruct((B, N, D), _f32)],
        compiler_params=pltpu.CompilerParams(
            dimension_semantics=("parallel", "parallel")),
    )(attn_raw, geom, qf, wattn, battn, wsm, bsm, gcq, bcq, wq, wp2, bp2)


def _mix2_body(cross_ref, cg_ref, qfeat_ref, wattn_ref, battn_ref, wcm_ref,
               bcm_ref, gffn_ref, bffn_ref, wfc1_ref, bfc1_ref, wfc2_ref,
               bfc2_ref, out_ref):
    cm = jnp.concatenate([cross_ref[0, h] for h in range(H)], axis=-1)
    c = _dot(cm, wattn_ref[...]) + battn_ref[0]
    c2 = _dot(c, wcm_ref[:D]) + _dot(cg_ref[0], wcm_ref[D:]) + bcm_ref[0]
    qf2 = qfeat_ref[0] + c2
    f = _ln(qf2, gffn_ref[0], bffn_ref[0])
    z = _dot(f, wfc1_ref[...]) + bfc1_ref[0]
    h1 = z * 0.5 * (1.0 + lax.erf(z * (2.0 ** -0.5)))
    y = _dot(h1, wfc2_ref[...]) + bfc2_ref[0]
    out_ref[0] = qf2 + y


def _mix2_call(cross_raw, cg, qfeat, wattn, battn, wcm, bcm, gffn, bffn,
               wfc1, bfc1, wfc2, bfc2):
    tok = pl.BlockSpec((1, BLK, D), lambda b_, i: (b_, i, 0))
    hd = pl.BlockSpec((1, H, BLK, DK), lambda b_, i: (b_, 0, i, 0))
    vec = pl.BlockSpec((1, D), lambda b_, i: (0, 0))
    vec2 = pl.BlockSpec((1, 2 * D), lambda b_, i: (0, 0))
    sq = pl.BlockSpec((D, D), lambda b_, i: (0, 0))
    dbl = pl.BlockSpec((2 * D, D), lambda b_, i: (0, 0))
    wide = pl.BlockSpec((D, 2 * D), lambda b_, i: (0, 0))
    return pl.pallas_call(
        _mix2_body,
        grid=(B, N // BLK),
        in_specs=[hd, tok, tok, sq, vec, dbl, vec, vec, vec, wide, vec2,
                  dbl, vec],
        out_specs=tok,
        out_shape=jax.ShapeDtypeStruct((B, N, D), _f32),
        compiler_params=pltpu.CompilerParams(
            dimension_semantics=("parallel", "parallel")),
    )(cross_raw, cg, qfeat, wattn, battn, wcm, bcm, gffn, bffn, wfc1, bfc1,
      wfc2, bfc2)


def kernel(query_points, key_points, g_in, b_in, W_qkv, W_attn, b_attn,
           W_fc1, b_fc1, W_fc2, b_fc2, g_ffn, b_ffn, W_p1, b_p1, W_p2, b_p2,
           W_sm, b_sm, W_cm, b_cm, g_cq, b_cq, g_ck, b_ck, Wq, Wk, Wv):
    qc = query_points[:, :3, :]
    kc = key_points[:, :3, :]
    qf = query_points[:, 3:, :].transpose(0, 2, 1)
    kf = key_points[:, 3:, :].transpose(0, 2, 1)
    qct = qc.transpose(0, 2, 1)

    def r2(v):
        return v.reshape(1, -1)

    qkv, p1, t1 = _pre_call(qf, r2(g_in), r2(b_in), W_qkv, W_p1, r2(b_p1))
    attn_raw = _attn_call(qkv, qkv, 6, 12, N)

    idx1 = _topk_call(qct, qc, N)
    gath1 = _gather_rows(p1.reshape(B * N, D), idx1.reshape(-1))
    geom = _combine_call(gath1.reshape(B, N, K, D), t1)

    qfeat, q2, t2 = _mix1_call(attn_raw, geom, qf, W_attn, r2(b_attn),
                               W_sm, r2(b_sm), r2(g_cq), r2(b_cq), Wq,
                               W_p2, r2(b_p2))

    kv2, p2 = _keys_call(kf, r2(g_ck), r2(b_ck), Wk, Wv, W_p2)
    cross_raw = _attn_call(q2, kv2, 0, 6, M)

    idx2 = _topk_call(qct, kc, M)
    gath2 = _gather_rows(p2.reshape(B * M, D), idx2.reshape(-1))
    cg = _combine_call(gath2.reshape(B, N, K, D), t2)

    out_feat = _mix2_call(cross_raw, cg, qfeat, W_attn, r2(b_attn), W_cm,
                          r2(b_cm), r2(g_ffn), r2(b_ffn), W_fc1, r2(b_fc1),
                          W_fc2, r2(b_fc2))
    return jnp.concatenate([qc, out_feat.transpose(0, 2, 1)], axis=1)


# speedup vs baseline: 1.1556x; 1.1556x over previous
import functools

import jax
import jax.numpy as jnp
from jax import lax
from jax.experimental import pallas as pl
from jax.experimental.pallas import tpu as pltpu
from jax.experimental.pallas import tpu_sc as plsc

B, N, M, D, H, K = 2, 2048, 2048, 384, 6, 8
DK = D // H
BLK = 256
QBLK = 512
SC_CORES, SC_SUBCORES = 2, 16
NW = SC_CORES * SC_SUBCORES
GCHUNK = 128

_f32 = jnp.float32


def _ln(x, g, b, eps=1e-5):
    mu = jnp.mean(x, axis=-1, keepdims=True)
    xc = x - mu
    var = jnp.mean(xc * xc, axis=-1, keepdims=True)
    return xc * lax.rsqrt(var + eps) * g + b


def _dot(a, b):
    return jnp.dot(a.astype(jnp.bfloat16), b.astype(jnp.bfloat16),
                   preferred_element_type=_f32)


def _pre_body(qf_ref, g_ref, b_ref, wqkv_ref, wp1_ref, bp1_ref,
              qkv_ref, p1_ref, t1_ref):
    nf = _ln(qf_ref[0], g_ref[0], b_ref[0])
    qkv = _dot(nf, wqkv_ref[...])
    for j in range(3 * H):
        qkv_ref[0, j] = qkv[:, j * DK:(j + 1) * DK]
    wa = wp1_ref[:D]
    wb = wp1_ref[D:]
    p1_ref[0] = _dot(nf, wa)
    t1_ref[0] = _dot(nf, wb - wa) + bp1_ref[0]


def _pre_call(qf, g, b, wqkv, wp1, bp1):
    return pl.pallas_call(
        _pre_body,
        grid=(B, N // BLK),
        in_specs=[
            pl.BlockSpec((1, BLK, D), lambda b_, i: (b_, i, 0)),
            pl.BlockSpec((1, D), lambda b_, i: (0, 0)),
            pl.BlockSpec((1, D), lambda b_, i: (0, 0)),
            pl.BlockSpec((D, 3 * D), lambda b_, i: (0, 0)),
            pl.BlockSpec((2 * D, D), lambda b_, i: (0, 0)),
            pl.BlockSpec((1, D), lambda b_, i: (0, 0)),
        ],
        out_specs=[
            pl.BlockSpec((1, 3 * H, BLK, DK), lambda b_, i: (b_, 0, i, 0)),
            pl.BlockSpec((1, BLK, D), lambda b_, i: (b_, i, 0)),
            pl.BlockSpec((1, BLK, D), lambda b_, i: (b_, i, 0)),
        ],
        out_shape=[
            jax.ShapeDtypeStruct((B, 3 * H, N, DK), _f32),
            jax.ShapeDtypeStruct((B, N, D), _f32),
            jax.ShapeDtypeStruct((B, N, D), _f32),
        ],
        compiler_params=pltpu.CompilerParams(
            dimension_semantics=("parallel", "parallel")),
    )(qf, g, b, wqkv, wp1, bp1)


def _keys_body(kf_ref, g_ref, b_ref, wk_ref, wv_ref, wp2_ref, kv_ref, p2_ref):
    nk = _ln(kf_ref[0], g_ref[0], b_ref[0])
    k2 = _dot(nk, wk_ref[...])
    v2 = _dot(nk, wv_ref[...])
    for h in range(H):
        kv_ref[0, h] = k2[:, h * DK:(h + 1) * DK]
        kv_ref[0, H + h] = v2[:, h * DK:(h + 1) * DK]
    p2_ref[0] = _dot(nk, wp2_ref[:D])


def _keys_call(kf, g, b, wk, wv, wp2):
    return pl.pallas_call(
        _keys_body,
        grid=(B, M // BLK),
        in_specs=[
            pl.BlockSpec((1, BLK, D), lambda b_, i: (b_, i, 0)),
            pl.BlockSpec((1, D), lambda b_, i: (0, 0)),
            pl.BlockSpec((1, D), lambda b_, i: (0, 0)),
            pl.BlockSpec((D, D), lambda b_, i: (0, 0)),
            pl.BlockSpec((D, D), lambda b_, i: (0, 0)),
            pl.BlockSpec((2 * D, D), lambda b_, i: (0, 0)),
        ],
        out_specs=[
            pl.BlockSpec((1, 2 * H, BLK, DK), lambda b_, i: (b_, 0, i, 0)),
            pl.BlockSpec((1, BLK, D), lambda b_, i: (b_, i, 0)),
        ],
        out_shape=[
            jax.ShapeDtypeStruct((B, 2 * H, M, DK), _f32),
            jax.ShapeDtypeStruct((B, M, D), _f32),
        ],
        compiler_params=pltpu.CompilerParams(
            dimension_semantics=("parallel", "parallel")),
    )(kf, g, b, wk, wv, wp2)


def _attn_body(q_ref, k_ref, v_ref, o_ref):
    q = (q_ref[0, 0] * (DK ** -0.5)).astype(jnp.bfloat16)
    k = k_ref[0, 0].astype(jnp.bfloat16)
    s = lax.dot_general(q, k, (((1,), (1,)), ((), ())),
                        preferred_element_type=_f32)
    e = jnp.exp(s)
    p = e * (1.0 / jnp.sum(e, axis=-1, keepdims=True))
    o_ref[0, 0] = _dot(p, v_ref[0, 0])


def _attn_call(q_arr, kv_arr, k_off, v_off, nkv):
    return pl.pallas_call(
        _attn_body,
        grid=(B, N // QBLK, H),
        in_specs=[
            pl.BlockSpec((1, 1, QBLK, DK), lambda b_, i, h: (b_, h, i, 0)),
            pl.BlockSpec((1, 1, nkv, DK),
                         lambda b_, i, h: (b_, h + k_off, 0, 0)),
            pl.BlockSpec((1, 1, nkv, DK),
                         lambda b_, i, h: (b_, h + v_off, 0, 0)),
        ],
        out_specs=pl.BlockSpec((1, 1, QBLK, DK),
                               lambda b_, i, h: (b_, h, i, 0)),
        out_shape=jax.ShapeDtypeStruct((B, H, N, DK), _f32),
        compiler_params=pltpu.CompilerParams(
            dimension_semantics=("parallel", "parallel", "arbitrary")),
    )(q_arr, kv_arr, kv_arr)


def _topk_body(qc_ref, kc_ref, idx_ref, *, nkeys):
    b_ = pl.program_id(0)
    q = qc_ref[0]
    kx = kc_ref[0]
    qq = jnp.zeros((BLK, 1), _f32)
    kk = jnp.zeros((1, nkeys), _f32)
    for c in range(3):
        qc_col = q[:, c:c + 1]
        kc_row = kx[c:c + 1, :]
        qq = qq + qc_col * qc_col
        kk = kk + kc_row * kc_row
    qk = jnp.dot(q.astype(jnp.bfloat16), kx.astype(jnp.bfloat16),
                 preferred_element_type=_f32)
    d2 = qq + kk - 2.0 * qk
    iota = lax.broadcasted_iota(jnp.int32, (BLK, nkeys), 1)
    cols = []
    for _ in range(K):
        sel = jnp.argmin(d2, axis=1).astype(jnp.int32)[:, None]
        cols.append(sel)
        d2 = jnp.where(iota == sel, jnp.float32(jnp.inf), d2)
    idx_ref[0] = jnp.concatenate(cols, axis=1) + b_ * nkeys


def _topk_call(qct, kcoord, nkeys):
    body = functools.partial(_topk_body, nkeys=nkeys)
    return pl.pallas_call(
        body,
        grid=(B, N // BLK),
        in_specs=[
            pl.BlockSpec((1, BLK, 3), lambda b_, i: (b_, i, 0)),
            pl.BlockSpec((1, 3, nkeys), lambda b_, i: (b_, 0, 0)),
        ],
        out_specs=pl.BlockSpec((1, BLK, K), lambda b_, i: (b_, i, 0)),
        out_shape=jax.ShapeDtypeStruct((B, N, K), jnp.int32),
        compiler_params=pltpu.CompilerParams(
            dimension_semantics=("parallel", "parallel")),
    )(qct, kcoord)


@functools.lru_cache(maxsize=None)
def _make_sc_gather():
    mesh = plsc.VectorSubcoreMesh(core_axis_name="c", subcore_axis_name="s")

    @functools.partial(
        pl.kernel,
        out_type=jax.ShapeDtypeStruct((B * N * K, D), _f32),
        mesh=mesh,
        scratch_types=[
            pltpu.VMEM((GCHUNK,), jnp.int32),
            pltpu.VMEM((GCHUNK, D), _f32),
            pltpu.SemaphoreType.DMA,
        ],
    )
    def sc_gather(table_hbm, idx_hbm, out_hbm, idx_v, rows_v, sem):
        wid = lax.axis_index("s") * SC_CORES + lax.axis_index("c")
        per_w = (B * N * K) // NW
        base = wid * per_w
        for c in range(per_w // GCHUNK):
            off = base + c * GCHUNK
            pltpu.sync_copy(idx_hbm.at[pl.ds(off, GCHUNK)], idx_v)
            pltpu.async_copy(table_hbm.at[idx_v], rows_v, sem).wait()
            pltpu.sync_copy(rows_v, out_hbm.at[pl.ds(off, GCHUNK)])

    return sc_gather


def _gather_rows(table, idx):
    return _make_sc_gather()(table, idx)


def _combine(g_ref, t_ref):
    t = t_ref[0]
    acc = None
    for k in range(K):
        x = g_ref[0, :, k, :] + t
        y = jnp.maximum(x, 0.2 * x)
        acc = y if acc is None else jnp.maximum(acc, y)
    return acc


def _mix1_body(attn_ref, g1_ref, t1_ref, qf_ref, wattn_ref, battn_ref,
               wsm_ref, bsm_ref, gcq_ref, bcq_ref, wq_ref, wp2_ref, bp2_ref,
               qfeat_ref, q2_ref, t2_ref):
    geom = _combine(g1_ref, t1_ref)
    am = jnp.concatenate([attn_ref[0, h] for h in range(H)], axis=-1)
    a = _dot(am, wattn_ref[...]) + battn_ref[0]
    a2 = _dot(a, wsm_ref[:D]) + _dot(geom, wsm_ref[D:]) + bsm_ref[0]
    qfeat = a2 + qf_ref[0]
    qfeat_ref[0] = qfeat
    nq = _ln(qfeat, gcq_ref[0], bcq_ref[0])
    q2 = _dot(nq, wq_ref[...])
    for h in range(H):
        q2_ref[0, h] = q2[:, h * DK:(h + 1) * DK]
    wa = wp2_ref[:D]
    wb = wp2_ref[D:]
    t2_ref[0] = _dot(nq, wb - wa) + bp2_ref[0]


def _mix1_call(attn_raw, gath1, t1, qf, wattn, battn, wsm, bsm, gcq, bcq, wq,
               wp2, bp2):
    tok = pl.BlockSpec((1, BLK, D), lambda b_, i: (b_, i, 0))
    hd = pl.BlockSpec((1, H, BLK, DK), lambda b_, i: (b_, 0, i, 0))
    gsp = pl.BlockSpec((1, BLK, K, D), lambda b_, i: (b_, i, 0, 0))
    vec = pl.BlockSpec((1, D), lambda b_, i: (0, 0))
    sq = pl.BlockSpec((D, D), lambda b_, i: (0, 0))
    dbl = pl.BlockSpec((2 * D, D), lambda b_, i: (0, 0))
    return pl.pallas_call(
        _mix1_body,
        grid=(B, N // BLK),
        in_specs=[hd, gsp, tok, tok, sq, vec, dbl, vec, vec, vec, sq, dbl,
                  vec],
        out_specs=[tok, hd, tok],
        out_shape=[jax.ShapeDtypeStruct((B, N, D), _f32),
                   jax.ShapeDtypeStruct((B, H, N, DK), _f32),
                   jax.ShapeDtypeStruct((B, N, D), _f32)],
        compiler_params=pltpu.CompilerParams(
            dimension_semantics=("parallel", "parallel")),
    )(attn_raw, gath1, t1, qf, wattn, battn, wsm, bsm, gcq, bcq, wq, wp2, bp2)


def _mix2_body(cross_ref, g2_ref, t2_ref, qfeat_ref, wattn_ref, battn_ref,
               wcm_ref, bcm_ref, gffn_ref, bffn_ref, wfc1_ref, bfc1_ref,
               wfc2_ref, bfc2_ref, out_ref):
    cg = _combine(g2_ref, t2_ref)
    cm = jnp.concatenate([cross_ref[0, h] for h in range(H)], axis=-1)
    c = _dot(cm, wattn_ref[...]) + battn_ref[0]
    c2 = _dot(c, wcm_ref[:D]) + _dot(cg, wcm_ref[D:]) + bcm_ref[0]
    qf2 = qfeat_ref[0] + c2
    f = _ln(qf2, gffn_ref[0], bffn_ref[0])
    z = _dot(f, wfc1_ref[...]) + bfc1_ref[0]
    h1 = z * 0.5 * (1.0 + lax.erf(z * (2.0 ** -0.5)))
    y = _dot(h1, wfc2_ref[...]) + bfc2_ref[0]
    out_ref[0] = qf2 + y


def _mix2_call(cross_raw, gath2, t2, qfeat, wattn, battn, wcm, bcm, gffn,
               bffn, wfc1, bfc1, wfc2, bfc2):
    tok = pl.BlockSpec((1, BLK, D), lambda b_, i: (b_, i, 0))
    hd = pl.BlockSpec((1, H, BLK, DK), lambda b_, i: (b_, 0, i, 0))
    gsp = pl.BlockSpec((1, BLK, K, D), lambda b_, i: (b_, i, 0, 0))
    vec = pl.BlockSpec((1, D), lambda b_, i: (0, 0))
    vec2 = pl.BlockSpec((1, 2 * D), lambda b_, i: (0, 0))
    sq = pl.BlockSpec((D, D), lambda b_, i: (0, 0))
    dbl = pl.BlockSpec((2 * D, D), lambda b_, i: (0, 0))
    wide = pl.BlockSpec((D, 2 * D), lambda b_, i: (0, 0))
    return pl.pallas_call(
        _mix2_body,
        grid=(B, N // BLK),
        in_specs=[hd, gsp, tok, tok, sq, vec, dbl, vec, vec, vec, wide,
                  vec2, dbl, vec],
        out_specs=tok,
        out_shape=jax.ShapeDtypeStruct((B, N, D), _f32),
        compiler_params=pltpu.CompilerParams(
            dimension_semantics=("parallel", "parallel")),
    )(cross_raw, gath2, t2, qfeat, wattn, battn, wcm, bcm, gffn, bffn,
      wfc1, bfc1, wfc2, bfc2)


def kernel(query_points, key_points, g_in, b_in, W_qkv, W_attn, b_attn,
           W_fc1, b_fc1, W_fc2, b_fc2, g_ffn, b_ffn, W_p1, b_p1, W_p2, b_p2,
           W_sm, b_sm, W_cm, b_cm, g_cq, b_cq, g_ck, b_ck, Wq, Wk, Wv):
    qc = query_points[:, :3, :]
    kc = key_points[:, :3, :]
    qf = query_points[:, 3:, :].transpose(0, 2, 1)
    kf = key_points[:, 3:, :].transpose(0, 2, 1)
    qct = qc.transpose(0, 2, 1)

    def r2(v):
        return v.reshape(1, -1)

    qkv, p1, t1 = _pre_call(qf, r2(g_in), r2(b_in), W_qkv, W_p1, r2(b_p1))
    attn_raw = _attn_call(qkv, qkv, 6, 12, N)

    idx1 = _topk_call(qct, qc, N)
    gath1 = _gather_rows(p1.reshape(B * N, D), idx1.reshape(-1))

    qfeat, q2, t2 = _mix1_call(attn_raw, gath1.reshape(B, N, K, D), t1, qf,
                               W_attn, r2(b_attn), W_sm, r2(b_sm), r2(g_cq),
                               r2(b_cq), Wq, W_p2, r2(b_p2))

    kv2, p2 = _keys_call(kf, r2(g_ck), r2(b_ck), Wk, Wv, W_p2)
    cross_raw = _attn_call(q2, kv2, 0, 6, M)

    idx2 = _topk_call(qct, kc, M)
    gath2 = _gather_rows(p2.reshape(B * M, D), idx2.reshape(-1))

    out_feat = _mix2_call(cross_raw, gath2.reshape(B, N, K, D), t2, qfeat,
                          W_attn, r2(b_attn), W_cm, r2(b_cm), r2(g_ffn),
                          r2(b_ffn), W_fc1, r2(b_fc1), W_fc2, r2(b_fc2))
    return jnp.concatenate([qc, out_feat.transpose(0, 2, 1)], axis=1)


# speedup vs baseline: 1.4352x; 1.2420x over previous
import functools

import jax
import jax.numpy as jnp
from jax import lax
from jax.experimental import pallas as pl
from jax.experimental.pallas import tpu as pltpu
from jax.experimental.pallas import tpu_sc as plsc

B, N, M, D, H, K = 2, 2048, 2048, 384, 6, 8
DK = D // H
BLK = 256
QBLK = 1024
SC_CORES, SC_SUBCORES = 2, 16
NW = SC_CORES * SC_SUBCORES
GCHUNK = 128

_f32 = jnp.float32


def _ln(x, g, b, eps=1e-5):
    mu = jnp.mean(x, axis=-1, keepdims=True)
    xc = x - mu
    var = jnp.mean(xc * xc, axis=-1, keepdims=True)
    return xc * lax.rsqrt(var + eps) * g + b


def _dot(a, b):
    return jnp.dot(a.astype(jnp.bfloat16), b.astype(jnp.bfloat16),
                   preferred_element_type=_f32)


def _pre_body(qf_ref, g_ref, b_ref, wqkv_ref, wp1_ref, bp1_ref,
              qkv_ref, p1_ref, t1_ref):
    nf = _ln(qf_ref[0], g_ref[0], b_ref[0])
    qkv = _dot(nf, wqkv_ref[...])
    for j in range(3 * H):
        qkv_ref[0, j] = qkv[:, j * DK:(j + 1) * DK]
    wa = wp1_ref[:D]
    wb = wp1_ref[D:]
    p1_ref[0] = _dot(nf, wa)
    t1_ref[0] = _dot(nf, wb - wa) + bp1_ref[0]


def _pre_call(qf, g, b, wqkv, wp1, bp1):
    return pl.pallas_call(
        _pre_body,
        grid=(B, N // BLK),
        in_specs=[
            pl.BlockSpec((1, BLK, D), lambda b_, i: (b_, i, 0)),
            pl.BlockSpec((1, D), lambda b_, i: (0, 0)),
            pl.BlockSpec((1, D), lambda b_, i: (0, 0)),
            pl.BlockSpec((D, 3 * D), lambda b_, i: (0, 0)),
            pl.BlockSpec((2 * D, D), lambda b_, i: (0, 0)),
            pl.BlockSpec((1, D), lambda b_, i: (0, 0)),
        ],
        out_specs=[
            pl.BlockSpec((1, 3 * H, BLK, DK), lambda b_, i: (b_, 0, i, 0)),
            pl.BlockSpec((1, BLK, D), lambda b_, i: (b_, i, 0)),
            pl.BlockSpec((1, BLK, D), lambda b_, i: (b_, i, 0)),
        ],
        out_shape=[
            jax.ShapeDtypeStruct((B, 3 * H, N, DK), _f32),
            jax.ShapeDtypeStruct((B, N, D), _f32),
            jax.ShapeDtypeStruct((B, N, D), _f32),
        ],
        compiler_params=pltpu.CompilerParams(
            dimension_semantics=("parallel", "parallel")),
    )(qf, g, b, wqkv, wp1, bp1)


def _keys_body(kf_ref, g_ref, b_ref, wk_ref, wv_ref, wp2_ref, kv_ref, p2_ref):
    nk = _ln(kf_ref[0], g_ref[0], b_ref[0])
    k2 = _dot(nk, wk_ref[...])
    v2 = _dot(nk, wv_ref[...])
    for h in range(H):
        kv_ref[0, h] = k2[:, h * DK:(h + 1) * DK]
        kv_ref[0, H + h] = v2[:, h * DK:(h + 1) * DK]
    p2_ref[0] = _dot(nk, wp2_ref[:D])


def _keys_call(kf, g, b, wk, wv, wp2):
    return pl.pallas_call(
        _keys_body,
        grid=(B, M // BLK),
        in_specs=[
            pl.BlockSpec((1, BLK, D), lambda b_, i: (b_, i, 0)),
            pl.BlockSpec((1, D), lambda b_, i: (0, 0)),
            pl.BlockSpec((1, D), lambda b_, i: (0, 0)),
            pl.BlockSpec((D, D), lambda b_, i: (0, 0)),
            pl.BlockSpec((D, D), lambda b_, i: (0, 0)),
            pl.BlockSpec((2 * D, D), lambda b_, i: (0, 0)),
        ],
        out_specs=[
            pl.BlockSpec((1, 2 * H, BLK, DK), lambda b_, i: (b_, 0, i, 0)),
            pl.BlockSpec((1, BLK, D), lambda b_, i: (b_, i, 0)),
        ],
        out_shape=[
            jax.ShapeDtypeStruct((B, 2 * H, M, DK), _f32),
            jax.ShapeDtypeStruct((B, M, D), _f32),
        ],
        compiler_params=pltpu.CompilerParams(
            dimension_semantics=("parallel", "parallel")),
    )(kf, g, b, wk, wv, wp2)


def _attn_body(q_ref, k_ref, v_ref, o_ref):
    q = (q_ref[0, 0] * (DK ** -0.5)).astype(jnp.bfloat16)
    k = k_ref[0, 0].astype(jnp.bfloat16)
    s = lax.dot_general(q, k, (((1,), (1,)), ((), ())),
                        preferred_element_type=_f32)
    e = jnp.exp(s)
    o = _dot(e, v_ref[0, 0])
    o_ref[0, 0] = o * (1.0 / jnp.sum(e, axis=-1, keepdims=True))


def _attn_call(q_arr, kv_arr, k_off, v_off, nkv):
    return pl.pallas_call(
        _attn_body,
        grid=(B, N // QBLK, H),
        in_specs=[
            pl.BlockSpec((1, 1, QBLK, DK), lambda b_, i, h: (b_, h, i, 0)),
            pl.BlockSpec((1, 1, nkv, DK),
                         lambda b_, i, h: (b_, h + k_off, 0, 0)),
            pl.BlockSpec((1, 1, nkv, DK),
                         lambda b_, i, h: (b_, h + v_off, 0, 0)),
        ],
        out_specs=pl.BlockSpec((1, 1, QBLK, DK),
                               lambda b_, i, h: (b_, h, i, 0)),
        out_shape=jax.ShapeDtypeStruct((B, H, N, DK), _f32),
        compiler_params=pltpu.CompilerParams(
            dimension_semantics=("parallel", "parallel", "arbitrary")),
    )(q_arr, kv_arr, kv_arr)


def _topk_body(qc_ref, kc_ref, idx_ref, *, nkeys):
    b_ = pl.program_id(0)
    q = qc_ref[0]
    kx = kc_ref[0]
    qq = jnp.zeros((BLK, 1), _f32)
    kk = jnp.zeros((1, nkeys), _f32)
    for c in range(3):
        qc_col = q[:, c:c + 1]
        kc_row = kx[c:c + 1, :]
        qq = qq + qc_col * qc_col
        kk = kk + kc_row * kc_row
    qk = jnp.dot(q.astype(jnp.bfloat16), kx.astype(jnp.bfloat16),
                 preferred_element_type=_f32)
    d2 = qq + kk - 2.0 * qk
    iota = lax.broadcasted_iota(jnp.int32, (BLK, nkeys), 1)
    cols = []
    for _ in range(K):
        sel = jnp.argmin(d2, axis=1).astype(jnp.int32)[:, None]
        cols.append(sel)
        d2 = jnp.where(iota == sel, jnp.float32(jnp.inf), d2)
    idx_ref[0] = jnp.concatenate(cols, axis=1) + b_ * nkeys


def _topk_call(qct, kcoord, nkeys):
    body = functools.partial(_topk_body, nkeys=nkeys)
    return pl.pallas_call(
        body,
        grid=(B, N // BLK),
        in_specs=[
            pl.BlockSpec((1, BLK, 3), lambda b_, i: (b_, i, 0)),
            pl.BlockSpec((1, 3, nkeys), lambda b_, i: (b_, 0, 0)),
        ],
        out_specs=pl.BlockSpec((1, BLK, K), lambda b_, i: (b_, i, 0)),
        out_shape=jax.ShapeDtypeStruct((B, N, K), jnp.int32),
        compiler_params=pltpu.CompilerParams(
            dimension_semantics=("parallel", "parallel")),
    )(qct, kcoord)


@functools.lru_cache(maxsize=None)
def _make_sc_gather():
    mesh = plsc.VectorSubcoreMesh(core_axis_name="c", subcore_axis_name="s")

    @functools.partial(
        pl.kernel,
        out_type=jax.ShapeDtypeStruct((B * N * K, D), _f32),
        mesh=mesh,
        scratch_types=[
            pltpu.VMEM((GCHUNK,), jnp.int32),
            pltpu.VMEM((GCHUNK, D), _f32),
            pltpu.SemaphoreType.DMA,
        ],
    )
    def sc_gather(table_hbm, idx_hbm, out_hbm, idx_v, rows_v, sem):
        wid = lax.axis_index("s") * SC_CORES + lax.axis_index("c")
        per_w = (B * N * K) // NW
        base = wid * per_w
        for c in range(per_w // GCHUNK):
            off = base + c * GCHUNK
            pltpu.sync_copy(idx_hbm.at[pl.ds(off, GCHUNK)], idx_v)
            pltpu.async_copy(table_hbm.at[idx_v], rows_v, sem).wait()
            pltpu.sync_copy(rows_v, out_hbm.at[pl.ds(off, GCHUNK)])

    return sc_gather


def _gather_rows(table, idx):
    return _make_sc_gather()(table, idx)


def _combine(g_ref, t_ref):
    t = t_ref[0]
    acc = None
    for k in range(K):
        x = g_ref[0, :, k, :] + t
        y = jnp.maximum(x, 0.2 * x)
        acc = y if acc is None else jnp.maximum(acc, y)
    return acc


def _mix1_body(attn_ref, g1_ref, t1_ref, qf_ref, wattn_ref, battn_ref,
               wsm_ref, bsm_ref, gcq_ref, bcq_ref, wq_ref, wp2_ref, bp2_ref,
               qfeat_ref, q2_ref, t2_ref):
    geom = _combine(g1_ref, t1_ref)
    am = jnp.concatenate([attn_ref[0, h] for h in range(H)], axis=-1)
    a = _dot(am, wattn_ref[...]) + battn_ref[0]
    a2 = _dot(a, wsm_ref[:D]) + _dot(geom, wsm_ref[D:]) + bsm_ref[0]
    qfeat = a2 + qf_ref[0]
    qfeat_ref[0] = qfeat
    nq = _ln(qfeat, gcq_ref[0], bcq_ref[0])
    q2 = _dot(nq, wq_ref[...])
    for h in range(H):
        q2_ref[0, h] = q2[:, h * DK:(h + 1) * DK]
    wa = wp2_ref[:D]
    wb = wp2_ref[D:]
    t2_ref[0] = _dot(nq, wb - wa) + bp2_ref[0]


def _mix1_call(attn_raw, gath1, t1, qf, wattn, battn, wsm, bsm, gcq, bcq, wq,
               wp2, bp2):
    tok = pl.BlockSpec((1, BLK, D), lambda b_, i: (b_, i, 0))
    hd = pl.BlockSpec((1, H, BLK, DK), lambda b_, i: (b_, 0, i, 0))
    gsp = pl.BlockSpec((1, BLK, K, D), lambda b_, i: (b_, i, 0, 0))
    vec = pl.BlockSpec((1, D), lambda b_, i: (0, 0))
    sq = pl.BlockSpec((D, D), lambda b_, i: (0, 0))
    dbl = pl.BlockSpec((2 * D, D), lambda b_, i: (0, 0))
    return pl.pallas_call(
        _mix1_body,
        grid=(B, N // BLK),
        in_specs=[hd, gsp, tok, tok, sq, vec, dbl, vec, vec, vec, sq, dbl,
                  vec],
        out_specs=[tok, hd, tok],
        out_shape=[jax.ShapeDtypeStruct((B, N, D), _f32),
                   jax.ShapeDtypeStruct((B, H, N, DK), _f32),
                   jax.ShapeDtypeStruct((B, N, D), _f32)],
        compiler_params=pltpu.CompilerParams(
            dimension_semantics=("parallel", "parallel")),
    )(attn_raw, gath1, t1, qf, wattn, battn, wsm, bsm, gcq, bcq, wq, wp2, bp2)


def _mix2_body(cross_ref, g2_ref, t2_ref, qfeat_ref, wattn_ref, battn_ref,
               wcm_ref, bcm_ref, gffn_ref, bffn_ref, wfc1_ref, bfc1_ref,
               wfc2_ref, bfc2_ref, out_ref):
    cg = _combine(g2_ref, t2_ref)
    cm = jnp.concatenate([cross_ref[0, h] for h in range(H)], axis=-1)
    c = _dot(cm, wattn_ref[...]) + battn_ref[0]
    c2 = _dot(c, wcm_ref[:D]) + _dot(cg, wcm_ref[D:]) + bcm_ref[0]
    qf2 = qfeat_ref[0] + c2
    f = _ln(qf2, gffn_ref[0], bffn_ref[0])
    z = _dot(f, wfc1_ref[...]) + bfc1_ref[0]
    h1 = z * 0.5 * (1.0 + lax.erf(z * (2.0 ** -0.5)))
    y = _dot(h1, wfc2_ref[...]) + bfc2_ref[0]
    out_ref[0] = qf2 + y


def _mix2_call(cross_raw, gath2, t2, qfeat, wattn, battn, wcm, bcm, gffn,
               bffn, wfc1, bfc1, wfc2, bfc2):
    tok = pl.BlockSpec((1, BLK, D), lambda b_, i: (b_, i, 0))
    hd = pl.BlockSpec((1, H, BLK, DK), lambda b_, i: (b_, 0, i, 0))
    gsp = pl.BlockSpec((1, BLK, K, D), lambda b_, i: (b_, i, 0, 0))
    vec = pl.BlockSpec((1, D), lambda b_, i: (0, 0))
    vec2 = pl.BlockSpec((1, 2 * D), lambda b_, i: (0, 0))
    sq = pl.BlockSpec((D, D), lambda b_, i: (0, 0))
    dbl = pl.BlockSpec((2 * D, D), lambda b_, i: (0, 0))
    wide = pl.BlockSpec((D, 2 * D), lambda b_, i: (0, 0))
    return pl.pallas_call(
        _mix2_body,
        grid=(B, N // BLK),
        in_specs=[hd, gsp, tok, tok, sq, vec, dbl, vec, vec, vec, wide,
                  vec2, dbl, vec],
        out_specs=tok,
        out_shape=jax.ShapeDtypeStruct((B, N, D), _f32),
        compiler_params=pltpu.CompilerParams(
            dimension_semantics=("parallel", "parallel")),
    )(cross_raw, gath2, t2, qfeat, wattn, battn, wcm, bcm, gffn, bffn,
      wfc1, bfc1, wfc2, bfc2)


def kernel(query_points, key_points, g_in, b_in, W_qkv, W_attn, b_attn,
           W_fc1, b_fc1, W_fc2, b_fc2, g_ffn, b_ffn, W_p1, b_p1, W_p2, b_p2,
           W_sm, b_sm, W_cm, b_cm, g_cq, b_cq, g_ck, b_ck, Wq, Wk, Wv):
    qc = query_points[:, :3, :]
    kc = key_points[:, :3, :]
    qf = query_points[:, 3:, :].transpose(0, 2, 1)
    kf = key_points[:, 3:, :].transpose(0, 2, 1)
    qct = qc.transpose(0, 2, 1)

    def r2(v):
        return v.reshape(1, -1)

    qkv, p1, t1 = _pre_call(qf, r2(g_in), r2(b_in), W_qkv, W_p1, r2(b_p1))
    attn_raw = _attn_call(qkv, qkv, 6, 12, N)

    idx1 = _topk_call(qct, qc, N)
    gath1 = _gather_rows(p1.reshape(B * N, D), idx1.reshape(-1))

    qfeat, q2, t2 = _mix1_call(attn_raw, gath1.reshape(B, N, K, D), t1, qf,
                               W_attn, r2(b_attn), W_sm, r2(b_sm), r2(g_cq),
                               r2(b_cq), Wq, W_p2, r2(b_p2))

    kv2, p2 = _keys_call(kf, r2(g_ck), r2(b_ck), Wk, Wv, W_p2)
    cross_raw = _attn_call(q2, kv2, 0, 6, M)

    idx2 = _topk_call(qct, kc, M)
    gath2 = _gather_rows(p2.reshape(B * M, D), idx2.reshape(-1))

    out_feat = _mix2_call(cross_raw, gath2.reshape(B, N, K, D), t2, qfeat,
                          W_attn, r2(b_attn), W_cm, r2(b_cm), r2(g_ffn),
                          r2(b_ffn), W_fc1, r2(b_fc1), W_fc2, r2(b_fc2))
    return jnp.concatenate([qc, out_feat.transpose(0, 2, 1)], axis=1)


# speedup vs baseline: 1.5509x; 1.0807x over previous
import functools

import jax
import jax.numpy as jnp
from jax import lax
from jax.experimental import pallas as pl
from jax.experimental.pallas import tpu as pltpu
from jax.experimental.pallas import tpu_sc as plsc

B, N, M, D, H, K = 2, 2048, 2048, 384, 6, 8
DK = D // H
BLK = 256
QBLK = 1024
SC_CORES, SC_SUBCORES = 2, 16
NW = SC_CORES * SC_SUBCORES
GCHUNK = 128

_f32 = jnp.float32


def _ln(x, g, b, eps=1e-5):
    mu = jnp.mean(x, axis=-1, keepdims=True)
    xc = x - mu
    var = jnp.mean(xc * xc, axis=-1, keepdims=True)
    return xc * lax.rsqrt(var + eps) * g + b


def _dot(a, b):
    return jnp.dot(a.astype(jnp.bfloat16), b.astype(jnp.bfloat16),
                   preferred_element_type=_f32)


def _pre_body(qf_ref, g_ref, b_ref, wqkv_ref, wp1_ref, bp1_ref,
              qkv_ref, p1_ref, t1_ref):
    nf = _ln(qf_ref[0], g_ref[0], b_ref[0])
    qkv = _dot(nf, wqkv_ref[...])
    for j in range(3 * H):
        qkv_ref[0, j] = qkv[:, j * DK:(j + 1) * DK]
    wa = wp1_ref[:D]
    wb = wp1_ref[D:]
    p1_ref[0] = _dot(nf, wa)
    t1_ref[0] = _dot(nf, wb - wa) + bp1_ref[0]


def _pre_call(qf, g, b, wqkv, wp1, bp1):
    return pl.pallas_call(
        _pre_body,
        grid=(B, N // BLK),
        in_specs=[
            pl.BlockSpec((1, BLK, D), lambda b_, i: (b_, i, 0)),
            pl.BlockSpec((1, D), lambda b_, i: (0, 0)),
            pl.BlockSpec((1, D), lambda b_, i: (0, 0)),
            pl.BlockSpec((D, 3 * D), lambda b_, i: (0, 0)),
            pl.BlockSpec((2 * D, D), lambda b_, i: (0, 0)),
            pl.BlockSpec((1, D), lambda b_, i: (0, 0)),
        ],
        out_specs=[
            pl.BlockSpec((1, 3 * H, BLK, DK), lambda b_, i: (b_, 0, i, 0)),
            pl.BlockSpec((1, BLK, D), lambda b_, i: (b_, i, 0)),
            pl.BlockSpec((1, BLK, D), lambda b_, i: (b_, i, 0)),
        ],
        out_shape=[
            jax.ShapeDtypeStruct((B, 3 * H, N, DK), _f32),
            jax.ShapeDtypeStruct((B, N, D), _f32),
            jax.ShapeDtypeStruct((B, N, D), _f32),
        ],
        compiler_params=pltpu.CompilerParams(
            dimension_semantics=("parallel", "parallel")),
    )(qf, g, b, wqkv, wp1, bp1)


def _keys_body(kf_ref, g_ref, b_ref, wk_ref, wv_ref, wp2_ref, kv_ref, p2_ref):
    nk = _ln(kf_ref[0], g_ref[0], b_ref[0])
    k2 = _dot(nk, wk_ref[...])
    v2 = _dot(nk, wv_ref[...])
    for h in range(H):
        kv_ref[0, h] = k2[:, h * DK:(h + 1) * DK]
        kv_ref[0, H + h] = v2[:, h * DK:(h + 1) * DK]
    p2_ref[0] = _dot(nk, wp2_ref[:D])


def _keys_call(kf, g, b, wk, wv, wp2):
    return pl.pallas_call(
        _keys_body,
        grid=(B, M // BLK),
        in_specs=[
            pl.BlockSpec((1, BLK, D), lambda b_, i: (b_, i, 0)),
            pl.BlockSpec((1, D), lambda b_, i: (0, 0)),
            pl.BlockSpec((1, D), lambda b_, i: (0, 0)),
            pl.BlockSpec((D, D), lambda b_, i: (0, 0)),
            pl.BlockSpec((D, D), lambda b_, i: (0, 0)),
            pl.BlockSpec((2 * D, D), lambda b_, i: (0, 0)),
        ],
        out_specs=[
            pl.BlockSpec((1, 2 * H, BLK, DK), lambda b_, i: (b_, 0, i, 0)),
            pl.BlockSpec((1, BLK, D), lambda b_, i: (b_, i, 0)),
        ],
        out_shape=[
            jax.ShapeDtypeStruct((B, 2 * H, M, DK), _f32),
            jax.ShapeDtypeStruct((B, M, D), _f32),
        ],
        compiler_params=pltpu.CompilerParams(
            dimension_semantics=("parallel", "parallel")),
    )(kf, g, b, wk, wv, wp2)


def _attn_body(q_ref, k_ref, v_ref, o_ref):
    q = (q_ref[0, 0] * (DK ** -0.5)).astype(jnp.bfloat16)
    k = k_ref[0, 0].astype(jnp.bfloat16)
    s = lax.dot_general(q, k, (((1,), (1,)), ((), ())),
                        preferred_element_type=_f32)
    e = jnp.exp(s)
    o = _dot(e, v_ref[0, 0])
    o_ref[0, 0] = o * (1.0 / jnp.sum(e, axis=-1, keepdims=True))


def _attn_call(q_arr, kv_arr, k_off, v_off, nkv):
    return pl.pallas_call(
        _attn_body,
        grid=(B, N // QBLK, H),
        in_specs=[
            pl.BlockSpec((1, 1, QBLK, DK), lambda b_, i, h: (b_, h, i, 0)),
            pl.BlockSpec((1, 1, nkv, DK),
                         lambda b_, i, h: (b_, h + k_off, 0, 0)),
            pl.BlockSpec((1, 1, nkv, DK),
                         lambda b_, i, h: (b_, h + v_off, 0, 0)),
        ],
        out_specs=pl.BlockSpec((1, 1, QBLK, DK),
                               lambda b_, i, h: (b_, h, i, 0)),
        out_shape=jax.ShapeDtypeStruct((B, H, N, DK), _f32),
        compiler_params=pltpu.CompilerParams(
            dimension_semantics=("parallel", "parallel", "arbitrary")),
    )(q_arr, kv_arr, kv_arr)


def _topk_body(qc_ref, kc_ref, idx_ref, *, nkeys):
    b_ = pl.program_id(0)
    q = qc_ref[0]
    kx = kc_ref[0]
    qq = jnp.zeros((BLK, 1), _f32)
    kk = jnp.zeros((1, nkeys), _f32)
    for c in range(3):
        qc_col = q[:, c:c + 1]
        kc_row = kx[c:c + 1, :]
        qq = qq + qc_col * qc_col
        kk = kk + kc_row * kc_row
    qk = jnp.dot(q.astype(jnp.bfloat16), kx.astype(jnp.bfloat16),
                 preferred_element_type=_f32)
    d2 = qq + kk - 2.0 * qk
    iota = lax.broadcasted_iota(jnp.int32, (BLK, nkeys), 1)
    cols = []
    for _ in range(K):
        sel = jnp.argmin(d2, axis=1).astype(jnp.int32)[:, None]
        cols.append(sel)
        d2 = jnp.where(iota == sel, jnp.float32(jnp.inf), d2)
    idx_ref[0] = jnp.concatenate(cols, axis=1) + b_ * nkeys


def _topk_call(qct, kcoord, nkeys):
    body = functools.partial(_topk_body, nkeys=nkeys)
    return pl.pallas_call(
        body,
        grid=(B, N // BLK),
        in_specs=[
            pl.BlockSpec((1, BLK, 3), lambda b_, i: (b_, i, 0)),
            pl.BlockSpec((1, 3, nkeys), lambda b_, i: (b_, 0, 0)),
        ],
        out_specs=pl.BlockSpec((1, BLK, K), lambda b_, i: (b_, i, 0)),
        out_shape=jax.ShapeDtypeStruct((B, N, K), jnp.int32),
        compiler_params=pltpu.CompilerParams(
            dimension_semantics=("parallel", "parallel")),
    )(qct, kcoord)


@functools.lru_cache(maxsize=None)
def _make_sc_geom():
    mesh = plsc.VectorSubcoreMesh(core_axis_name="c", subcore_axis_name="s")
    rows_per_w = B * N // NW
    qc_ = 16

    @functools.partial(
        pl.kernel,
        out_type=jax.ShapeDtypeStruct((B * N, D), _f32),
        mesh=mesh,
        scratch_types=[
            pltpu.VMEM((qc_ * K,), jnp.int32),
            pltpu.VMEM((qc_ * K, D), _f32),
            pltpu.VMEM((qc_, D), _f32),
            pltpu.VMEM((qc_, D), _f32),
            pltpu.SemaphoreType.DMA,
        ],
    )
    def sc_geom(table_hbm, idx_hbm, t_hbm, out_hbm, idx_v, rows_v, t_v,
                out_v, sem):
        wid = lax.axis_index("s") * SC_CORES + lax.axis_index("c")
        base = wid * rows_per_w
        for c in range(rows_per_w // qc_):
            qoff = base + c * qc_
            pltpu.sync_copy(idx_hbm.at[pl.ds(qoff * K, qc_ * K)], idx_v)
            pltpu.sync_copy(t_hbm.at[pl.ds(qoff, qc_)], t_v)
            pltpu.async_copy(table_hbm.at[idx_v], rows_v, sem).wait()

            def qbody(q, carry):
                for j in range(D // 16):
                    sl = pl.ds(j * 16, 16)
                    t = t_v[q, sl]
                    acc = None
                    for k in range(K):
                        x = rows_v[q * K + k, sl] + t
                        y = jnp.maximum(x, 0.2 * x)
                        acc = y if acc is None else jnp.maximum(acc, y)
                    out_v[q, sl] = acc
                return carry

            lax.fori_loop(0, qc_, qbody, 0)
            pltpu.sync_copy(out_v, out_hbm.at[pl.ds(qoff, qc_)])

    return sc_geom


def _geom_rows(table, idx, t):
    return _make_sc_geom()(table, idx, t)


def _mix1_body(attn_ref, geom_ref, qf_ref, wattn_ref, battn_ref,
               wsm_ref, bsm_ref, gcq_ref, bcq_ref, wq_ref, wp2_ref, bp2_ref,
               qfeat_ref, q2_ref, t2_ref):
    geom = geom_ref[0]
    am = jnp.concatenate([attn_ref[0, h] for h in range(H)], axis=-1)
    a = _dot(am, wattn_ref[...]) + battn_ref[0]
    a2 = _dot(a, wsm_ref[:D]) + _dot(geom, wsm_ref[D:]) + bsm_ref[0]
    qfeat = a2 + qf_ref[0]
    qfeat_ref[0] = qfeat
    nq = _ln(qfeat, gcq_ref[0], bcq_ref[0])
    q2 = _dot(nq, wq_ref[...])
    for h in range(H):
        q2_ref[0, h] = q2[:, h * DK:(h + 1) * DK]
    wa = wp2_ref[:D]
    wb = wp2_ref[D:]
    t2_ref[0] = _dot(nq, wb - wa) + bp2_ref[0]


def _mix1_call(attn_raw, geom, qf, wattn, battn, wsm, bsm, gcq, bcq, wq,
               wp2, bp2):
    tok = pl.BlockSpec((1, BLK, D), lambda b_, i: (b_, i, 0))
    hd = pl.BlockSpec((1, H, BLK, DK), lambda b_, i: (b_, 0, i, 0))
    vec = pl.BlockSpec((1, D), lambda b_, i: (0, 0))
    sq = pl.BlockSpec((D, D), lambda b_, i: (0, 0))
    dbl = pl.BlockSpec((2 * D, D), lambda b_, i: (0, 0))
    return pl.pallas_call(
        _mix1_body,
        grid=(B, N // BLK),
        in_specs=[hd, tok, tok, sq, vec, dbl, vec, vec, vec, sq, dbl,
                  vec],
        out_specs=[tok, hd, tok],
        out_shape=[jax.ShapeDtypeStruct((B, N, D), _f32),
                   jax.ShapeDtypeStruct((B, H, N, DK), _f32),
                   jax.ShapeDtypeStruct((B, N, D), _f32)],
        compiler_params=pltpu.CompilerParams(
            dimension_semantics=("parallel", "parallel")),
    )(attn_raw, geom, qf, wattn, battn, wsm, bsm, gcq, bcq, wq, wp2, bp2)


def _mix2_body(cross_ref, cg_ref, qfeat_ref, wattn_ref, battn_ref,
               wcm_ref, bcm_ref, gffn_ref, bffn_ref, wfc1_ref, bfc1_ref,
               wfc2_ref, bfc2_ref, out_ref):
    cg = cg_ref[0]
    cm = jnp.concatenate([cross_ref[0, h] for h in range(H)], axis=-1)
    c = _dot(cm, wattn_ref[...]) + battn_ref[0]
    c2 = _dot(c, wcm_ref[:D]) + _dot(cg, wcm_ref[D:]) + bcm_ref[0]
    qf2 = qfeat_ref[0] + c2
    f = _ln(qf2, gffn_ref[0], bffn_ref[0])
    z = _dot(f, wfc1_ref[...]) + bfc1_ref[0]
    h1 = z * 0.5 * (1.0 + lax.erf(z * (2.0 ** -0.5)))
    y = _dot(h1, wfc2_ref[...]) + bfc2_ref[0]
    out_ref[0] = qf2 + y


def _mix2_call(cross_raw, cg, qfeat, wattn, battn, wcm, bcm, gffn,
               bffn, wfc1, bfc1, wfc2, bfc2):
    tok = pl.BlockSpec((1, BLK, D), lambda b_, i: (b_, i, 0))
    hd = pl.BlockSpec((1, H, BLK, DK), lambda b_, i: (b_, 0, i, 0))
    vec = pl.BlockSpec((1, D), lambda b_, i: (0, 0))
    vec2 = pl.BlockSpec((1, 2 * D), lambda b_, i: (0, 0))
    sq = pl.BlockSpec((D, D), lambda b_, i: (0, 0))
    dbl = pl.BlockSpec((2 * D, D), lambda b_, i: (0, 0))
    wide = pl.BlockSpec((D, 2 * D), lambda b_, i: (0, 0))
    return pl.pallas_call(
        _mix2_body,
        grid=(B, N // BLK),
        in_specs=[hd, tok, tok, sq, vec, dbl, vec, vec, vec, wide,
                  vec2, dbl, vec],
        out_specs=tok,
        out_shape=jax.ShapeDtypeStruct((B, N, D), _f32),
        compiler_params=pltpu.CompilerParams(
            dimension_semantics=("parallel", "parallel")),
    )(cross_raw, cg, qfeat, wattn, battn, wcm, bcm, gffn, bffn,
      wfc1, bfc1, wfc2, bfc2)


def kernel(query_points, key_points, g_in, b_in, W_qkv, W_attn, b_attn,
           W_fc1, b_fc1, W_fc2, b_fc2, g_ffn, b_ffn, W_p1, b_p1, W_p2, b_p2,
           W_sm, b_sm, W_cm, b_cm, g_cq, b_cq, g_ck, b_ck, Wq, Wk, Wv):
    qc = query_points[:, :3, :]
    kc = key_points[:, :3, :]
    qf = query_points[:, 3:, :].transpose(0, 2, 1)
    kf = key_points[:, 3:, :].transpose(0, 2, 1)
    qct = qc.transpose(0, 2, 1)

    def r2(v):
        return v.reshape(1, -1)

    qkv, p1, t1 = _pre_call(qf, r2(g_in), r2(b_in), W_qkv, W_p1, r2(b_p1))
    attn_raw = _attn_call(qkv, qkv, 6, 12, N)

    idx1 = _topk_call(qct, qc, N)
    geom = _geom_rows(p1.reshape(B * N, D), idx1.reshape(-1),
                      t1.reshape(B * N, D))

    qfeat, q2, t2 = _mix1_call(attn_raw, geom.reshape(B, N, D), qf,
                               W_attn, r2(b_attn), W_sm, r2(b_sm), r2(g_cq),
                               r2(b_cq), Wq, W_p2, r2(b_p2))

    kv2, p2 = _keys_call(kf, r2(g_ck), r2(b_ck), Wk, Wv, W_p2)
    cross_raw = _attn_call(q2, kv2, 0, 6, M)

    idx2 = _topk_call(qct, kc, M)
    cg = _geom_rows(p2.reshape(B * M, D), idx2.reshape(-1),
                    t2.reshape(B * N, D))

    out_feat = _mix2_call(cross_raw, cg.reshape(B, N, D), qfeat,
                          W_attn, r2(b_attn), W_cm, r2(b_cm), r2(g_ffn),
                          r2(b_ffn), W_fc1, r2(b_fc1), W_fc2, r2(b_fc2))
    return jnp.concatenate([qc, out_feat.transpose(0, 2, 1)], axis=1)


# speedup vs baseline: 1.5749x; 1.0154x over previous
import functools

import jax
import jax.numpy as jnp
from jax import lax
from jax.experimental import pallas as pl
from jax.experimental.pallas import tpu as pltpu
from jax.experimental.pallas import tpu_sc as plsc

B, N, M, D, H, K = 2, 2048, 2048, 384, 6, 8
DK = D // H
BLK = 256
QBLK = 1024
SC_CORES, SC_SUBCORES = 2, 16
NW = SC_CORES * SC_SUBCORES
GCHUNK = 128
TBLK = 512

_f32 = jnp.float32


def _ln(x, g, b, eps=1e-5):
    mu = jnp.mean(x, axis=-1, keepdims=True)
    xc = x - mu
    var = jnp.mean(xc * xc, axis=-1, keepdims=True)
    return xc * lax.rsqrt(var + eps) * g + b


def _dot(a, b):
    return jnp.dot(a.astype(jnp.bfloat16), b.astype(jnp.bfloat16),
                   preferred_element_type=_f32)


def _pre_body(qf_ref, g_ref, b_ref, wqkv_ref, wp1_ref, bp1_ref,
              qkv_ref, p1_ref, t1_ref):
    nf = _ln(qf_ref[0], g_ref[0], b_ref[0])
    qkv = _dot(nf, wqkv_ref[...])
    for j in range(3 * H):
        qkv_ref[0, j] = qkv[:, j * DK:(j + 1) * DK]
    wa = wp1_ref[:D]
    wb = wp1_ref[D:]
    p1_ref[0] = _dot(nf, wa)
    t1_ref[0] = _dot(nf, wb - wa) + bp1_ref[0]


def _pre_call(qf, g, b, wqkv, wp1, bp1):
    return pl.pallas_call(
        _pre_body,
        grid=(B, N // BLK),
        in_specs=[
            pl.BlockSpec((1, BLK, D), lambda b_, i: (b_, i, 0)),
            pl.BlockSpec((1, D), lambda b_, i: (0, 0)),
            pl.BlockSpec((1, D), lambda b_, i: (0, 0)),
            pl.BlockSpec((D, 3 * D), lambda b_, i: (0, 0)),
            pl.BlockSpec((2 * D, D), lambda b_, i: (0, 0)),
            pl.BlockSpec((1, D), lambda b_, i: (0, 0)),
        ],
        out_specs=[
            pl.BlockSpec((1, 3 * H, BLK, DK), lambda b_, i: (b_, 0, i, 0)),
            pl.BlockSpec((1, BLK, D), lambda b_, i: (b_, i, 0)),
            pl.BlockSpec((1, BLK, D), lambda b_, i: (b_, i, 0)),
        ],
        out_shape=[
            jax.ShapeDtypeStruct((B, 3 * H, N, DK), _f32),
            jax.ShapeDtypeStruct((B, N, D), _f32),
            jax.ShapeDtypeStruct((B, N, D), _f32),
        ],
        compiler_params=pltpu.CompilerParams(
            dimension_semantics=("parallel", "parallel")),
    )(qf, g, b, wqkv, wp1, bp1)


def _keys_body(kf_ref, g_ref, b_ref, wk_ref, wv_ref, wp2_ref, kv_ref, p2_ref):
    nk = _ln(kf_ref[0], g_ref[0], b_ref[0])
    k2 = _dot(nk, wk_ref[...])
    v2 = _dot(nk, wv_ref[...])
    for h in range(H):
        kv_ref[0, h] = k2[:, h * DK:(h + 1) * DK]
        kv_ref[0, H + h] = v2[:, h * DK:(h + 1) * DK]
    p2_ref[0] = _dot(nk, wp2_ref[:D])


def _keys_call(kf, g, b, wk, wv, wp2):
    return pl.pallas_call(
        _keys_body,
        grid=(B, M // BLK),
        in_specs=[
            pl.BlockSpec((1, BLK, D), lambda b_, i: (b_, i, 0)),
            pl.BlockSpec((1, D), lambda b_, i: (0, 0)),
            pl.BlockSpec((1, D), lambda b_, i: (0, 0)),
            pl.BlockSpec((D, D), lambda b_, i: (0, 0)),
            pl.BlockSpec((D, D), lambda b_, i: (0, 0)),
            pl.BlockSpec((2 * D, D), lambda b_, i: (0, 0)),
        ],
        out_specs=[
            pl.BlockSpec((1, 2 * H, BLK, DK), lambda b_, i: (b_, 0, i, 0)),
            pl.BlockSpec((1, BLK, D), lambda b_, i: (b_, i, 0)),
        ],
        out_shape=[
            jax.ShapeDtypeStruct((B, 2 * H, M, DK), _f32),
            jax.ShapeDtypeStruct((B, M, D), _f32),
        ],
        compiler_params=pltpu.CompilerParams(
            dimension_semantics=("parallel", "parallel")),
    )(kf, g, b, wk, wv, wp2)


def _attn_body(q_ref, k_ref, v_ref, o_ref):
    q = (q_ref[0, 0] * (DK ** -0.5)).astype(jnp.bfloat16)
    k = k_ref[0, 0].astype(jnp.bfloat16)
    s = lax.dot_general(q, k, (((1,), (1,)), ((), ())),
                        preferred_element_type=_f32)
    e = jnp.exp(s)
    o = _dot(e, v_ref[0, 0])
    o_ref[0, 0] = o * (1.0 / jnp.sum(e, axis=-1, keepdims=True))


def _attn_call(q_arr, kv_arr, k_off, v_off, nkv):
    return pl.pallas_call(
        _attn_body,
        grid=(B, N // QBLK, H),
        in_specs=[
            pl.BlockSpec((1, 1, QBLK, DK), lambda b_, i, h: (b_, h, i, 0)),
            pl.BlockSpec((1, 1, nkv, DK),
                         lambda b_, i, h: (b_, h + k_off, 0, 0)),
            pl.BlockSpec((1, 1, nkv, DK),
                         lambda b_, i, h: (b_, h + v_off, 0, 0)),
        ],
        out_specs=pl.BlockSpec((1, 1, QBLK, DK),
                               lambda b_, i, h: (b_, h, i, 0)),
        out_shape=jax.ShapeDtypeStruct((B, H, N, DK), _f32),
        compiler_params=pltpu.CompilerParams(
            dimension_semantics=("parallel", "parallel", "arbitrary")),
    )(q_arr, kv_arr, kv_arr)


def _topk_body(qc_ref, kc_ref, idx_ref, *, nkeys):
    b_ = pl.program_id(0)
    q = qc_ref[0]
    kx = kc_ref[0]
    qq = jnp.zeros((TBLK, 1), _f32)
    kk = jnp.zeros((1, nkeys), _f32)
    for c in range(3):
        qc_col = q[:, c:c + 1]
        kc_row = kx[c:c + 1, :]
        qq = qq + qc_col * qc_col
        kk = kk + kc_row * kc_row
    qk = jnp.dot(q.astype(jnp.bfloat16), kx.astype(jnp.bfloat16),
                 preferred_element_type=_f32)
    d2 = qq + kk - 2.0 * qk
    iota = lax.broadcasted_iota(jnp.int32, (TBLK, nkeys), 1)
    cols = []
    for _ in range(K):
        sel = jnp.argmin(d2, axis=1).astype(jnp.int32)[:, None]
        cols.append(sel)
        d2 = jnp.where(iota == sel, jnp.float32(jnp.inf), d2)
    idx_ref[0] = jnp.concatenate(cols, axis=1) + b_ * nkeys


def _topk_call(qct, kcoord, nkeys):
    body = functools.partial(_topk_body, nkeys=nkeys)
    return pl.pallas_call(
        body,
        grid=(B, N // TBLK),
        in_specs=[
            pl.BlockSpec((1, TBLK, 3), lambda b_, i: (b_, i, 0)),
            pl.BlockSpec((1, 3, nkeys), lambda b_, i: (b_, 0, 0)),
        ],
        out_specs=pl.BlockSpec((1, TBLK, K), lambda b_, i: (b_, i, 0)),
        out_shape=jax.ShapeDtypeStruct((B, N, K), jnp.int32),
        compiler_params=pltpu.CompilerParams(
            dimension_semantics=("parallel", "parallel")),
    )(qct, kcoord)


@functools.lru_cache(maxsize=None)
def _make_sc_geom():
    mesh = plsc.VectorSubcoreMesh(core_axis_name="c", subcore_axis_name="s")
    rows_per_w = B * N // NW
    qc_ = 16

    @functools.partial(
        pl.kernel,
        out_type=jax.ShapeDtypeStruct((B * N, D), _f32),
        mesh=mesh,
        scratch_types=[
            pltpu.VMEM((qc_ * K,), jnp.int32),
            pltpu.VMEM((qc_ * K, D), _f32),
            pltpu.VMEM((qc_, D), _f32),
            pltpu.VMEM((qc_, D), _f32),
            pltpu.SemaphoreType.DMA,
        ],
    )
    def sc_geom(table_hbm, idx_hbm, t_hbm, out_hbm, idx_v, rows_v, t_v,
                out_v, sem):
        wid = lax.axis_index("s") * SC_CORES + lax.axis_index("c")
        base = wid * rows_per_w
        for c in range(rows_per_w // qc_):
            qoff = base + c * qc_
            pltpu.sync_copy(idx_hbm.at[pl.ds(qoff * K, qc_ * K)], idx_v)
            pltpu.sync_copy(t_hbm.at[pl.ds(qoff, qc_)], t_v)
            pltpu.async_copy(table_hbm.at[idx_v], rows_v, sem).wait()

            def qbody(q, carry):
                for j in range(D // 16):
                    sl = pl.ds(j * 16, 16)
                    t = t_v[q, sl]
                    acc = None
                    for k in range(K):
                        x = rows_v[q * K + k, sl] + t
                        y = jnp.maximum(x, 0.2 * x)
                        acc = y if acc is None else jnp.maximum(acc, y)
                    out_v[q, sl] = acc
                return carry

            lax.fori_loop(0, qc_, qbody, 0)
            pltpu.sync_copy(out_v, out_hbm.at[pl.ds(qoff, qc_)])

    return sc_geom


def _geom_rows(table, idx, t):
    return _make_sc_geom()(table, idx, t)


def _mix1_body(attn_ref, geom_ref, qf_ref, wattn_ref, battn_ref,
               wsm_ref, bsm_ref, gcq_ref, bcq_ref, wq_ref, wp2_ref, bp2_ref,
               qfeat_ref, q2_ref, t2_ref):
    geom = geom_ref[0]
    am = jnp.concatenate([attn_ref[0, h] for h in range(H)], axis=-1)
    a = _dot(am, wattn_ref[...]) + battn_ref[0]
    a2 = _dot(a, wsm_ref[:D]) + _dot(geom, wsm_ref[D:]) + bsm_ref[0]
    qfeat = a2 + qf_ref[0]
    qfeat_ref[0] = qfeat
    nq = _ln(qfeat, gcq_ref[0], bcq_ref[0])
    q2 = _dot(nq, wq_ref[...])
    for h in range(H):
        q2_ref[0, h] = q2[:, h * DK:(h + 1) * DK]
    wa = wp2_ref[:D]
    wb = wp2_ref[D:]
    t2_ref[0] = _dot(nq, wb - wa) + bp2_ref[0]


def _mix1_call(attn_raw, geom, qf, wattn, battn, wsm, bsm, gcq, bcq, wq,
               wp2, bp2):
    tok = pl.BlockSpec((1, BLK, D), lambda b_, i: (b_, i, 0))
    hd = pl.BlockSpec((1, H, BLK, DK), lambda b_, i: (b_, 0, i, 0))
    vec = pl.BlockSpec((1, D), lambda b_, i: (0, 0))
    sq = pl.BlockSpec((D, D), lambda b_, i: (0, 0))
    dbl = pl.BlockSpec((2 * D, D), lambda b_, i: (0, 0))
    return pl.pallas_call(
        _mix1_body,
        grid=(B, N // BLK),
        in_specs=[hd, tok, tok, sq, vec, dbl, vec, vec, vec, sq, dbl,
                  vec],
        out_specs=[tok, hd, tok],
        out_shape=[jax.ShapeDtypeStruct((B, N, D), _f32),
                   jax.ShapeDtypeStruct((B, H, N, DK), _f32),
                   jax.ShapeDtypeStruct((B, N, D), _f32)],
        compiler_params=pltpu.CompilerParams(
            dimension_semantics=("parallel", "parallel")),
    )(attn_raw, geom, qf, wattn, battn, wsm, bsm, gcq, bcq, wq, wp2, bp2)


def _mix2_body(cross_ref, cg_ref, qfeat_ref, wattn_ref, battn_ref,
               wcm_ref, bcm_ref, gffn_ref, bffn_ref, wfc1_ref, bfc1_ref,
               wfc2_ref, bfc2_ref, out_ref):
    cg = cg_ref[0]
    cm = jnp.concatenate([cross_ref[0, h] for h in range(H)], axis=-1)
    c = _dot(cm, wattn_ref[...]) + battn_ref[0]
    c2 = _dot(c, wcm_ref[:D]) + _dot(cg, wcm_ref[D:]) + bcm_ref[0]
    qf2 = qfeat_ref[0] + c2
    f = _ln(qf2, gffn_ref[0], bffn_ref[0])
    z = _dot(f, wfc1_ref[...]) + bfc1_ref[0]
    h1 = z * 0.5 * (1.0 + lax.erf(z * (2.0 ** -0.5)))
    y = _dot(h1, wfc2_ref[...]) + bfc2_ref[0]
    out_ref[0] = qf2 + y


def _mix2_call(cross_raw, cg, qfeat, wattn, battn, wcm, bcm, gffn,
               bffn, wfc1, bfc1, wfc2, bfc2):
    tok = pl.BlockSpec((1, BLK, D), lambda b_, i: (b_, i, 0))
    hd = pl.BlockSpec((1, H, BLK, DK), lambda b_, i: (b_, 0, i, 0))
    vec = pl.BlockSpec((1, D), lambda b_, i: (0, 0))
    vec2 = pl.BlockSpec((1, 2 * D), lambda b_, i: (0, 0))
    sq = pl.BlockSpec((D, D), lambda b_, i: (0, 0))
    dbl = pl.BlockSpec((2 * D, D), lambda b_, i: (0, 0))
    wide = pl.BlockSpec((D, 2 * D), lambda b_, i: (0, 0))
    return pl.pallas_call(
        _mix2_body,
        grid=(B, N // BLK),
        in_specs=[hd, tok, tok, sq, vec, dbl, vec, vec, vec, wide,
                  vec2, dbl, vec],
        out_specs=tok,
        out_shape=jax.ShapeDtypeStruct((B, N, D), _f32),
        compiler_params=pltpu.CompilerParams(
            dimension_semantics=("parallel", "parallel")),
    )(cross_raw, cg, qfeat, wattn, battn, wcm, bcm, gffn, bffn,
      wfc1, bfc1, wfc2, bfc2)


def kernel(query_points, key_points, g_in, b_in, W_qkv, W_attn, b_attn,
           W_fc1, b_fc1, W_fc2, b_fc2, g_ffn, b_ffn, W_p1, b_p1, W_p2, b_p2,
           W_sm, b_sm, W_cm, b_cm, g_cq, b_cq, g_ck, b_ck, Wq, Wk, Wv):
    qc = query_points[:, :3, :]
    kc = key_points[:, :3, :]
    qf = query_points[:, 3:, :].transpose(0, 2, 1)
    kf = key_points[:, 3:, :].transpose(0, 2, 1)
    qct = qc.transpose(0, 2, 1)

    def r2(v):
        return v.reshape(1, -1)

    qkv, p1, t1 = _pre_call(qf, r2(g_in), r2(b_in), W_qkv, W_p1, r2(b_p1))
    attn_raw = _attn_call(qkv, qkv, 6, 12, N)

    idx1 = _topk_call(qct, qc, N)
    geom = _geom_rows(p1.reshape(B * N, D), idx1.reshape(-1),
                      t1.reshape(B * N, D))

    qfeat, q2, t2 = _mix1_call(attn_raw, geom.reshape(B, N, D), qf,
                               W_attn, r2(b_attn), W_sm, r2(b_sm), r2(g_cq),
                               r2(b_cq), Wq, W_p2, r2(b_p2))

    kv2, p2 = _keys_call(kf, r2(g_ck), r2(b_ck), Wk, Wv, W_p2)
    cross_raw = _attn_call(q2, kv2, 0, 6, M)

    idx2 = _topk_call(qct, kc, M)
    cg = _geom_rows(p2.reshape(B * M, D), idx2.reshape(-1),
                    t2.reshape(B * N, D))

    out_feat = _mix2_call(cross_raw, cg.reshape(B, N, D), qfeat,
                          W_attn, r2(b_attn), W_cm, r2(b_cm), r2(g_ffn),
                          r2(b_ffn), W_fc1, r2(b_fc1), W_fc2, r2(b_fc2))
    return jnp.concatenate([qc, out_feat.transpose(0, 2, 1)], axis=1)
